# baseline (speedup 1.0000x reference)
import jax
import jax.numpy as jnp
from jax.experimental import pallas as pl
from jax.experimental.pallas import tpu as pltpu

B, S, D = 1, 2048, 1024
H, KVH, DH = 16, 8, 64
E, K, FF, R = 8, 2, 2816, 16
SCALE = 32.0 / 16.0
EPS = 1e-5
SBLK = 256
NBLK = S // SBLK
ABLK = 512
NABLK = S // ABLK
ER = E * R
NREP = H // KVH
TBLK = 512
LOG2E = 1.4426950408889634


def _rmsnorm(x, w):
    return x * jax.lax.rsqrt(jnp.mean(x * x, axis=-1, keepdims=True) + EPS) * w


def _qkv_kernel(x_ref, cos_ref, sin_ref, wn_ref, wqkv_ref,
                q_ref, k_ref, v_ref):
    bf = jnp.bfloat16
    h = _rmsnorm(x_ref[...], wn_ref[...]).astype(bf)
    qkv = jnp.dot(h, wqkv_ref[...], preferred_element_type=jnp.float32)
    q = qkv[:, :H * DH]
    k = qkv[:, H * DH:(H + KVH) * DH]
    v = qkv[:, (H + KVH) * DH:]
    cos = cos_ref[...]
    sin = sin_ref[...]
    hw = DH // 2
    for i in range(H):
        qh = q[:, i * DH:(i + 1) * DH]
        rot = jnp.concatenate([-qh[:, hw:], qh[:, :hw]], axis=1)
        q_ref[i] = ((qh * cos + rot * sin) * (LOG2E / (DH ** 0.5))).astype(bf)
    for i in range(KVH):
        kh = k[:, i * DH:(i + 1) * DH]
        rot = jnp.concatenate([-kh[:, hw:], kh[:, :hw]], axis=1)
        k_ref[i] = (kh * cos + rot * sin).astype(bf)
        v_ref[i] = v[:, i * DH:(i + 1) * DH].astype(bf)


def _attn_kernel(q_ref, k_ref, v_ref, mask_ref, o_ref):
    i = pl.program_id(1)
    q = q_ref[0]
    m = jnp.full((ABLK, 1), -1e30, jnp.float32)
    l = jnp.zeros((ABLK, 1), jnp.float32)
    o = jnp.zeros((ABLK, DH), jnp.float32)
    for t in range(S // TBLK):
        k = k_ref[0, pl.ds(t * TBLK, TBLK), :]
        s = jax.lax.dot_general(q, k, (((1,), (1,)), ((), ())),
                                preferred_element_type=jnp.float32)
        s = s + mask_ref[pl.ds(i * ABLK, ABLK), pl.ds(t * TBLK, TBLK)]
        mn = jnp.maximum(m, jnp.max(s, axis=1, keepdims=True))
        e = jnp.exp2(s - mn)
        alpha = jnp.exp2(m - mn)
        l = l * alpha + jnp.sum(e, axis=1, keepdims=True)
        o = o * alpha + jnp.dot(e.astype(jnp.bfloat16),
                                v_ref[0, pl.ds(t * TBLK, TBLK), :],
                                preferred_element_type=jnp.float32)
        m = mn
    o_ref[0] = (o / l).astype(jnp.bfloat16)


def _post_kernel(attn_ref, x_ref, wo_ref, wn_ref, gw_ref, d2_ref, sn_ref, lg_ref):
    o = jnp.dot(attn_ref[0], wo_ref[pl.ds(0, DH), :],
                preferred_element_type=jnp.float32)
    for i in range(1, H):
        o = o + jnp.dot(attn_ref[i], wo_ref[pl.ds(i * DH, DH), :],
                        preferred_element_type=jnp.float32)
    d2 = x_ref[...] + o
    sn = _rmsnorm(d2, wn_ref[...])
    d2_ref[...] = d2
    sn_ref[...] = sn
    lg_ref[...] = jnp.dot(sn, gw_ref[...], preferred_element_type=jnp.float32)


def _route_kernel(lg_ref, oh0_ref, oh1_ref, wt0_ref, wt1_ref):
    lg = lg_ref[...]
    cm = jnp.max(lg, axis=0, keepdims=True)
    ex = jnp.exp(lg - cm)
    p = ex / jnp.sum(ex, axis=0, keepdims=True)
    idx = jax.lax.broadcasted_iota(jnp.int32, (S, E), 1)
    v0 = jnp.max(p, axis=1, keepdims=True)
    i0 = jnp.min(jnp.where(p >= v0, idx, E), axis=1, keepdims=True)
    oh0 = idx == i0
    pm = jnp.where(oh0, -jnp.inf, p)
    v1 = jnp.max(pm, axis=1, keepdims=True)
    i1 = jnp.min(jnp.where(pm >= v1, idx, E), axis=1, keepdims=True)
    oh1 = idx == i1
    tot = v0 + v1
    oh0_ref[...] = oh0.astype(jnp.float32)
    oh1_ref[...] = oh1.astype(jnp.float32)
    wt0_ref[...] = jnp.broadcast_to(v0 / tot, (S, E))
    wt1_ref[...] = jnp.broadcast_to(v1 / tot, (S, E))


def _moe_kernel(d2_ref, sn_ref, oh0_ref, oh1_ref, wt0_ref, wt1_ref,
                w13_ref, w2_ref, a1t_ref, a3t_ref, b1_ref, b3_ref,
                a2t_ref, b2_ref, out_ref):
    bf = jnp.bfloat16
    sn = sn_ref[...].astype(bf)
    c13 = jnp.dot(sn, w13_ref[...], preferred_element_type=jnp.float32)
    c1 = c13[:, :FF]
    c3 = c13[:, FF:]
    z1 = jnp.dot(c1.astype(bf), a1t_ref[...], preferred_element_type=jnp.float32)
    z3 = jnp.dot(c3.astype(bf), a3t_ref[...], preferred_element_type=jnp.float32)
    re = jax.lax.broadcasted_iota(jnp.int32, (E, ER), 0)
    le = jax.lax.broadcasted_iota(jnp.int32, (E, ER), 1) // R
    expm = (re == le).astype(jnp.float32)
    us = []
    ms = []
    for oh_ref in (oh0_ref, oh1_ref):
        m = jnp.dot(oh_ref[...], expm, preferred_element_type=jnp.float32)
        zm1 = (z1 * m).astype(bf)
        zm3 = (z3 * m).astype(bf)
        corr1 = jnp.dot(zm1, b1_ref[...], preferred_element_type=jnp.float32) * SCALE
        corr3 = jnp.dot(zm3, b3_ref[...], preferred_element_type=jnp.float32) * SCALE
        g = c1 + corr1
        us.append(((g * jax.lax.logistic(g)) * (c3 + corr3)).astype(bf))
        ms.append(m)
    u = jnp.concatenate(us, axis=0)
    mcat = jnp.concatenate(ms, axis=0)
    hs = jnp.dot(u, w2_ref[...], preferred_element_type=jnp.float32)
    y2 = jnp.dot(hs.astype(bf), a2t_ref[...], preferred_element_type=jnp.float32)
    y2m = (y2 * mcat).astype(bf)
    hs = hs + jnp.dot(y2m, b2_ref[...], preferred_element_type=jnp.float32) * SCALE
    out_ref[...] = (d2_ref[...] + hs[:SBLK] * wt0_ref[...][:, :1]
                    + hs[SBLK:] * wt1_ref[...][:, :1])


def kernel(data, mask, rope_cos, rope_sin, attn_norm_w, ffn_norm_w, wq, wk, wv,
           wo, gate_w, w1, w2, w3, lora_a1, lora_b1, lora_a3, lora_b3,
           lora_a2, lora_b2):
    x = data[0]
    anw = attn_norm_w.reshape(1, D)
    fnw = ffn_norm_w.reshape(1, D)
    bf = jnp.bfloat16
    a1t = lora_a1.reshape(ER, FF).T.astype(bf)
    a3t = lora_a3.reshape(ER, FF).T.astype(bf)
    b1 = lora_b1.transpose(0, 2, 1).reshape(ER, FF).astype(bf)
    b3 = lora_b3.transpose(0, 2, 1).reshape(ER, FF).astype(bf)
    a2t = lora_a2.reshape(ER, D).T.astype(bf)
    b2 = lora_b2.transpose(0, 2, 1).reshape(ER, D).astype(bf)
    wqkv = jnp.concatenate([wq, wk, wv], axis=1).astype(bf)
    wob = wo.astype(bf)
    w13 = jnp.concatenate([w1, w3], axis=1).astype(bf)
    w2b = w2.astype(bf)
    f32 = jnp.float32

    full = lambda shape: pl.BlockSpec(shape, lambda i: (0,) * len(shape))
    rows = lambda cols: pl.BlockSpec((SBLK, cols), lambda i: (i, 0))

    q3, k3, v3 = pl.pallas_call(
        _qkv_kernel,
        grid=(NBLK,),
        in_specs=[rows(D), rows(DH), rows(DH), full((1, D)),
                  full((D, (H + 2 * KVH) * DH))],
        out_specs=[pl.BlockSpec((H, SBLK, DH), lambda i: (0, i, 0)),
                   pl.BlockSpec((KVH, SBLK, DH), lambda i: (0, i, 0)),
                   pl.BlockSpec((KVH, SBLK, DH), lambda i: (0, i, 0))],
        out_shape=[jax.ShapeDtypeStruct((H, S, DH), bf),
                   jax.ShapeDtypeStruct((KVH, S, DH), bf),
                   jax.ShapeDtypeStruct((KVH, S, DH), bf)],
        compiler_params=pltpu.CompilerParams(
            dimension_semantics=("parallel",)),
    )(x, rope_cos, rope_sin, anw, wqkv)

    attn3 = pl.pallas_call(
        _attn_kernel,
        grid=(H, NABLK),
        in_specs=[pl.BlockSpec((1, ABLK, DH), lambda h, i: (h, i, 0)),
                  pl.BlockSpec((1, S, DH), lambda h, i: (h // NREP, 0, 0)),
                  pl.BlockSpec((1, S, DH), lambda h, i: (h // NREP, 0, 0)),
                  pl.BlockSpec((S, S), lambda h, i: (0, 0))],
        out_specs=pl.BlockSpec((1, ABLK, DH), lambda h, i: (h, i, 0)),
        out_shape=jax.ShapeDtypeStruct((H, S, DH), bf),
        compiler_params=pltpu.CompilerParams(
            vmem_limit_bytes=100 * 1024 * 1024,
            dimension_semantics=("parallel", "parallel")),
    )(q3, k3, v3, mask * jnp.float32(LOG2E))

    d2, sn, lg = pl.pallas_call(
        _post_kernel,
        grid=(NBLK,),
        in_specs=[pl.BlockSpec((H, SBLK, DH), lambda i: (0, i, 0)),
                  rows(D), full((H * DH, D)), full((1, D)), full((D, E))],
        out_specs=[rows(D), rows(D), rows(E)],
        out_shape=[jax.ShapeDtypeStruct((S, D), f32),
                   jax.ShapeDtypeStruct((S, D), f32),
                   jax.ShapeDtypeStruct((S, E), f32)],
        compiler_params=pltpu.CompilerParams(
            dimension_semantics=("parallel",)),
    )(attn3, x, wob, fnw, gate_w)

    oh0, oh1, wt0, wt1 = pl.pallas_call(
        _route_kernel,
        out_shape=[jax.ShapeDtypeStruct((S, E), f32)] * 4,
    )(lg)

    out = pl.pallas_call(
        _moe_kernel,
        grid=(NBLK,),
        in_specs=[rows(D), rows(D), rows(E), rows(E), rows(E), rows(E),
                  full((D, 2 * FF)), full((FF, D)),
                  full((FF, ER)), full((FF, ER)), full((ER, FF)),
                  full((ER, FF)), full((D, ER)), full((ER, D))],
        out_specs=rows(D),
        out_shape=jax.ShapeDtypeStruct((S, D), f32),
        compiler_params=pltpu.CompilerParams(
            vmem_limit_bytes=100 * 1024 * 1024,
            dimension_semantics=("parallel",)),
    )(d2, sn, oh0, oh1, wt0, wt1, w13, w2b, a1t, a3t, b1, b3, a2t, b2)

    return out[None]


# speedup vs baseline: 1.0367x; 1.0367x over previous
import jax
import jax.numpy as jnp
from jax.experimental import pallas as pl
from jax.experimental.pallas import tpu as pltpu

B, S, D = 1, 2048, 1024
H, KVH, DH = 16, 8, 64
E, K, FF, R = 8, 2, 2816, 16
SCALE = 32.0 / 16.0
EPS = 1e-5
SBLK = 256
NBLK = S // SBLK
ABLK = 512
NABLK = S // ABLK
ER = E * R
NREP = H // KVH
TBLK = 512
LOG2E = 1.4426950408889634


def _rmsnorm(x, w):
    return x * jax.lax.rsqrt(jnp.mean(x * x, axis=-1, keepdims=True) + EPS) * w


def _qkv_kernel(x_ref, cos_ref, sin_ref, wn_ref, wq_ref, wk_ref, wv_ref,
                q_ref, k_ref, v_ref):
    bf = jnp.bfloat16
    h = _rmsnorm(x_ref[...], wn_ref[...]).astype(bf)
    q = jnp.dot(h, wq_ref[...], preferred_element_type=jnp.float32)
    k = jnp.dot(h, wk_ref[...], preferred_element_type=jnp.float32)
    v = jnp.dot(h, wv_ref[...], preferred_element_type=jnp.float32)
    cos = cos_ref[...]
    sin = sin_ref[...]
    hw = DH // 2
    for i in range(H):
        qh = q[:, i * DH:(i + 1) * DH]
        rot = jnp.concatenate([-qh[:, hw:], qh[:, :hw]], axis=1)
        q_ref[i] = ((qh * cos + rot * sin) * (LOG2E / (DH ** 0.5))).astype(bf)
    for i in range(KVH):
        kh = k[:, i * DH:(i + 1) * DH]
        rot = jnp.concatenate([-kh[:, hw:], kh[:, :hw]], axis=1)
        k_ref[i] = (kh * cos + rot * sin).astype(bf)
        v_ref[i] = v[:, i * DH:(i + 1) * DH].astype(bf)


def _attn_kernel(q_ref, k_ref, v_ref, mask_ref, o_ref):
    i = pl.program_id(1)
    q = q_ref[0]
    m = jnp.full((ABLK, 1), -1e30, jnp.float32)
    l = jnp.zeros((ABLK, 1), jnp.float32)
    o = jnp.zeros((ABLK, DH), jnp.float32)
    for t in range(S // TBLK):
        k = k_ref[0, pl.ds(t * TBLK, TBLK), :]
        s = jax.lax.dot_general(q, k, (((1,), (1,)), ((), ())),
                                preferred_element_type=jnp.float32)
        s = s + mask_ref[pl.ds(i * ABLK, ABLK), pl.ds(t * TBLK, TBLK)]
        mn = jnp.maximum(m, jnp.max(s, axis=1, keepdims=True))
        e = jnp.exp2(s - mn)
        alpha = jnp.exp2(m - mn)
        l = l * alpha + jnp.sum(e, axis=1, keepdims=True)
        o = o * alpha + jnp.dot(e.astype(jnp.bfloat16),
                                v_ref[0, pl.ds(t * TBLK, TBLK), :],
                                preferred_element_type=jnp.float32)
        m = mn
    o_ref[0] = (o / l).astype(jnp.bfloat16)


def _post_kernel(attn_ref, x_ref, wo_ref, wn_ref, gw_ref, d2_ref, sn_ref, lg_ref):
    o = jnp.dot(attn_ref[0], wo_ref[pl.ds(0, DH), :],
                preferred_element_type=jnp.float32)
    for i in range(1, H):
        o = o + jnp.dot(attn_ref[i], wo_ref[pl.ds(i * DH, DH), :],
                        preferred_element_type=jnp.float32)
    d2 = x_ref[...] + o
    sn = _rmsnorm(d2, wn_ref[...])
    d2_ref[...] = d2
    sn_ref[...] = sn
    lg_ref[...] = jnp.dot(sn, gw_ref[...], preferred_element_type=jnp.float32)


def _route_kernel(lg_ref, oh0_ref, oh1_ref, wt0_ref, wt1_ref):
    lg = lg_ref[...]
    cm = jnp.max(lg, axis=0, keepdims=True)
    ex = jnp.exp(lg - cm)
    p = ex / jnp.sum(ex, axis=0, keepdims=True)
    idx = jax.lax.broadcasted_iota(jnp.int32, (S, E), 1)
    v0 = jnp.max(p, axis=1, keepdims=True)
    i0 = jnp.min(jnp.where(p >= v0, idx, E), axis=1, keepdims=True)
    oh0 = idx == i0
    pm = jnp.where(oh0, -jnp.inf, p)
    v1 = jnp.max(pm, axis=1, keepdims=True)
    i1 = jnp.min(jnp.where(pm >= v1, idx, E), axis=1, keepdims=True)
    oh1 = idx == i1
    tot = v0 + v1
    oh0_ref[...] = oh0.astype(jnp.float32)
    oh1_ref[...] = oh1.astype(jnp.float32)
    wt0_ref[...] = jnp.broadcast_to(v0 / tot, (S, E))
    wt1_ref[...] = jnp.broadcast_to(v1 / tot, (S, E))


def _moe_kernel(d2_ref, sn_ref, oh0_ref, oh1_ref, wt0_ref, wt1_ref,
                w1_ref, w3_ref, w2_ref, a1t_ref, a3t_ref, b1_ref, b3_ref,
                a2t_ref, b2_ref, out_ref):
    bf = jnp.bfloat16
    sn = sn_ref[...].astype(bf)
    c1 = jnp.dot(sn, w1_ref[...], preferred_element_type=jnp.float32)
    c3 = jnp.dot(sn, w3_ref[...], preferred_element_type=jnp.float32)
    c1b = c1.astype(bf)
    c3b = c3.astype(bf)
    z1 = jnp.dot(c1b, a1t_ref[...], preferred_element_type=jnp.float32)
    z3 = jnp.dot(c3b, a3t_ref[...], preferred_element_type=jnp.float32)
    re = jax.lax.broadcasted_iota(jnp.int32, (E, ER), 0)
    le = jax.lax.broadcasted_iota(jnp.int32, (E, ER), 1) // R
    expm = (re == le).astype(jnp.float32)
    acc = d2_ref[...]
    for oh_ref, wt_ref in ((oh0_ref, wt0_ref), (oh1_ref, wt1_ref)):
        m = jnp.dot(oh_ref[...], expm, preferred_element_type=jnp.float32)
        zm1 = (z1 * m).astype(bf)
        zm3 = (z3 * m).astype(bf)
        corr1 = jnp.dot(zm1, b1_ref[...], preferred_element_type=jnp.float32) * SCALE
        corr3 = jnp.dot(zm3, b3_ref[...], preferred_element_type=jnp.float32) * SCALE
        g = c1 + corr1
        u = ((g * jax.lax.logistic(g)) * (c3 + corr3)).astype(bf)
        hs = jnp.dot(u, w2_ref[...], preferred_element_type=jnp.float32)
        y2 = jnp.dot(hs.astype(bf), a2t_ref[...], preferred_element_type=jnp.float32)
        y2m = (y2 * m).astype(bf)
        hs = hs + jnp.dot(y2m, b2_ref[...], preferred_element_type=jnp.float32) * SCALE
        acc = acc + hs * wt_ref[...][:, :1]
    out_ref[...] = acc


def kernel(data, mask, rope_cos, rope_sin, attn_norm_w, ffn_norm_w, wq, wk, wv,
           wo, gate_w, w1, w2, w3, lora_a1, lora_b1, lora_a3, lora_b3,
           lora_a2, lora_b2):
    x = data[0]
    anw = attn_norm_w.reshape(1, D)
    fnw = ffn_norm_w.reshape(1, D)
    bf = jnp.bfloat16
    a1t = lora_a1.reshape(ER, FF).T.astype(bf)
    a3t = lora_a3.reshape(ER, FF).T.astype(bf)
    b1 = lora_b1.transpose(0, 2, 1).reshape(ER, FF).astype(bf)
    b3 = lora_b3.transpose(0, 2, 1).reshape(ER, FF).astype(bf)
    a2t = lora_a2.reshape(ER, D).T.astype(bf)
    b2 = lora_b2.transpose(0, 2, 1).reshape(ER, D).astype(bf)
    wqb = wq.astype(bf)
    wkb = wk.astype(bf)
    wvb = wv.astype(bf)
    wob = wo.astype(bf)
    w1b = w1.astype(bf)
    w2b = w2.astype(bf)
    w3b = w3.astype(bf)
    f32 = jnp.float32

    full = lambda shape: pl.BlockSpec(shape, lambda i: (0,) * len(shape))
    rows = lambda cols: pl.BlockSpec((SBLK, cols), lambda i: (i, 0))

    q3, k3, v3 = pl.pallas_call(
        _qkv_kernel,
        grid=(NBLK,),
        in_specs=[rows(D), rows(DH), rows(DH), full((1, D)),
                  full((D, H * DH)), full((D, KVH * DH)), full((D, KVH * DH))],
        out_specs=[pl.BlockSpec((H, SBLK, DH), lambda i: (0, i, 0)),
                   pl.BlockSpec((KVH, SBLK, DH), lambda i: (0, i, 0)),
                   pl.BlockSpec((KVH, SBLK, DH), lambda i: (0, i, 0))],
        out_shape=[jax.ShapeDtypeStruct((H, S, DH), bf),
                   jax.ShapeDtypeStruct((KVH, S, DH), bf),
                   jax.ShapeDtypeStruct((KVH, S, DH), bf)],
    )(x, rope_cos, rope_sin, anw, wqb, wkb, wvb)

    attn3 = pl.pallas_call(
        _attn_kernel,
        grid=(H, NABLK),
        in_specs=[pl.BlockSpec((1, ABLK, DH), lambda h, i: (h, i, 0)),
                  pl.BlockSpec((1, S, DH), lambda h, i: (h // NREP, 0, 0)),
                  pl.BlockSpec((1, S, DH), lambda h, i: (h // NREP, 0, 0)),
                  pl.BlockSpec((S, S), lambda h, i: (0, 0))],
        out_specs=pl.BlockSpec((1, ABLK, DH), lambda h, i: (h, i, 0)),
        out_shape=jax.ShapeDtypeStruct((H, S, DH), bf),
        compiler_params=pltpu.CompilerParams(vmem_limit_bytes=100 * 1024 * 1024),
    )(q3, k3, v3, mask * jnp.float32(LOG2E))

    d2, sn, lg = pl.pallas_call(
        _post_kernel,
        grid=(NBLK,),
        in_specs=[pl.BlockSpec((H, SBLK, DH), lambda i: (0, i, 0)),
                  rows(D), full((H * DH, D)), full((1, D)), full((D, E))],
        out_specs=[rows(D), rows(D), rows(E)],
        out_shape=[jax.ShapeDtypeStruct((S, D), f32),
                   jax.ShapeDtypeStruct((S, D), f32),
                   jax.ShapeDtypeStruct((S, E), f32)],
    )(attn3, x, wob, fnw, gate_w)

    oh0, oh1, wt0, wt1 = pl.pallas_call(
        _route_kernel,
        out_shape=[jax.ShapeDtypeStruct((S, E), f32)] * 4,
    )(lg)

    out = pl.pallas_call(
        _moe_kernel,
        grid=(NBLK,),
        in_specs=[rows(D), rows(D), rows(E), rows(E), rows(E), rows(E),
                  full((D, FF)), full((D, FF)), full((FF, D)),
                  full((FF, ER)), full((FF, ER)), full((ER, FF)),
                  full((ER, FF)), full((D, ER)), full((ER, D))],
        out_specs=rows(D),
        out_shape=jax.ShapeDtypeStruct((S, D), f32),
        compiler_params=pltpu.CompilerParams(vmem_limit_bytes=100 * 1024 * 1024),
    )(d2, sn, oh0, oh1, wt0, wt1, w1b, w3b, w2b, a1t, a3t, b1, b3, a2t, b2)

    return out[None]


# speedup vs baseline: 1.1168x; 1.0772x over previous
import jax
import jax.numpy as jnp
from jax.experimental import pallas as pl
from jax.experimental.pallas import tpu as pltpu

B, S, D = 1, 2048, 1024
H, KVH, DH = 16, 8, 64
E, K, FF, R = 8, 2, 2816, 16
SCALE = 32.0 / 16.0
EPS = 1e-5
SBLK = 256
NBLK = S // SBLK
ABLK = 1024
NABLK = S // ABLK
ER = E * R
NREP = H // KVH
TBLK = 512
LOG2E = 1.4426950408889634


def _rmsnorm(x, w):
    return x * jax.lax.rsqrt(jnp.mean(x * x, axis=-1, keepdims=True) + EPS) * w


def _qkv_kernel(x_ref, cos_ref, sin_ref, wn_ref, wq_ref, wk_ref, wv_ref,
                q_ref, k_ref, v_ref):
    bf = jnp.bfloat16
    h = _rmsnorm(x_ref[...], wn_ref[...]).astype(bf)
    q = jnp.dot(h, wq_ref[...], preferred_element_type=jnp.float32)
    k = jnp.dot(h, wk_ref[...], preferred_element_type=jnp.float32)
    v = jnp.dot(h, wv_ref[...], preferred_element_type=jnp.float32)
    cos = cos_ref[...]
    sin = sin_ref[...]
    hw = DH // 2
    for i in range(H):
        qh = q[:, i * DH:(i + 1) * DH]
        rot = jnp.concatenate([-qh[:, hw:], qh[:, :hw]], axis=1)
        q_ref[i] = ((qh * cos + rot * sin) * (LOG2E / (DH ** 0.5))).astype(bf)
    for i in range(KVH):
        kh = k[:, i * DH:(i + 1) * DH]
        rot = jnp.concatenate([-kh[:, hw:], kh[:, :hw]], axis=1)
        k_ref[i] = (kh * cos + rot * sin).astype(bf)
        v_ref[i] = v[:, i * DH:(i + 1) * DH].astype(bf)


def _attn_kernel(q_ref, k_ref, v_ref, mask_ref, o_ref):
    i = pl.program_id(1)
    q = q_ref[0]
    m = jnp.full((ABLK, 1), -1e30, jnp.float32)
    l = jnp.zeros((ABLK, 1), jnp.float32)
    o = jnp.zeros((ABLK, DH), jnp.float32)
    for t in range(S // TBLK):
        k = k_ref[0, pl.ds(t * TBLK, TBLK), :]
        s = jax.lax.dot_general(q, k, (((1,), (1,)), ((), ())),
                                preferred_element_type=jnp.float32)
        s = s + mask_ref[pl.ds(i * ABLK, ABLK), pl.ds(t * TBLK, TBLK)]
        mn = jnp.maximum(m, jnp.max(s, axis=1, keepdims=True))
        e = jnp.exp2(s - mn)
        alpha = jnp.exp2(m - mn)
        l = l * alpha + jnp.sum(e, axis=1, keepdims=True)
        o = o * alpha + jnp.dot(e.astype(jnp.bfloat16),
                                v_ref[0, pl.ds(t * TBLK, TBLK), :],
                                preferred_element_type=jnp.float32)
        m = mn
    o_ref[0] = (o / l).astype(jnp.bfloat16)


def _post_kernel(attn_ref, x_ref, wo_ref, wn_ref, gw_ref, d2_ref, sn_ref, lg_ref):
    o = jnp.dot(attn_ref[0], wo_ref[pl.ds(0, DH), :],
                preferred_element_type=jnp.float32)
    for i in range(1, H):
        o = o + jnp.dot(attn_ref[i], wo_ref[pl.ds(i * DH, DH), :],
                        preferred_element_type=jnp.float32)
    d2 = x_ref[...] + o
    sn = _rmsnorm(d2, wn_ref[...])
    d2_ref[...] = d2
    sn_ref[...] = sn
    lg_ref[...] = jnp.dot(sn, gw_ref[...], preferred_element_type=jnp.float32)


def _route_kernel(lg_ref, oh0_ref, oh1_ref, wt0_ref, wt1_ref):
    lg = lg_ref[...]
    cm = jnp.max(lg, axis=0, keepdims=True)
    ex = jnp.exp(lg - cm)
    p = ex / jnp.sum(ex, axis=0, keepdims=True)
    idx = jax.lax.broadcasted_iota(jnp.int32, (S, E), 1)
    v0 = jnp.max(p, axis=1, keepdims=True)
    i0 = jnp.min(jnp.where(p >= v0, idx, E), axis=1, keepdims=True)
    oh0 = idx == i0
    pm = jnp.where(oh0, -jnp.inf, p)
    v1 = jnp.max(pm, axis=1, keepdims=True)
    i1 = jnp.min(jnp.where(pm >= v1, idx, E), axis=1, keepdims=True)
    oh1 = idx == i1
    tot = v0 + v1
    oh0_ref[...] = oh0.astype(jnp.float32)
    oh1_ref[...] = oh1.astype(jnp.float32)
    wt0_ref[...] = jnp.broadcast_to(v0 / tot, (S, E))
    wt1_ref[...] = jnp.broadcast_to(v1 / tot, (S, E))


def _moe_kernel(d2_ref, sn_ref, oh0_ref, oh1_ref, wt0_ref, wt1_ref,
                w1_ref, w3_ref, w2_ref, a1t_ref, a3t_ref, b1_ref, b3_ref,
                a2t_ref, b2_ref, out_ref):
    bf = jnp.bfloat16
    sn = sn_ref[...].astype(bf)
    c1 = jnp.dot(sn, w1_ref[...], preferred_element_type=jnp.float32)
    c3 = jnp.dot(sn, w3_ref[...], preferred_element_type=jnp.float32)
    c1b = c1.astype(bf)
    c3b = c3.astype(bf)
    z1 = jnp.dot(c1b, a1t_ref[...], preferred_element_type=jnp.float32)
    z3 = jnp.dot(c3b, a3t_ref[...], preferred_element_type=jnp.float32)
    re = jax.lax.broadcasted_iota(jnp.int32, (E, ER), 0)
    le = jax.lax.broadcasted_iota(jnp.int32, (E, ER), 1) // R
    expm = (re == le).astype(jnp.float32)
    acc = d2_ref[...]
    for oh_ref, wt_ref in ((oh0_ref, wt0_ref), (oh1_ref, wt1_ref)):
        m = jnp.dot(oh_ref[...], expm, preferred_element_type=jnp.float32)
        zm1 = (z1 * m).astype(bf)
        zm3 = (z3 * m).astype(bf)
        corr1 = jnp.dot(zm1, b1_ref[...], preferred_element_type=jnp.float32) * SCALE
        corr3 = jnp.dot(zm3, b3_ref[...], preferred_element_type=jnp.float32) * SCALE
        g = c1 + corr1
        u = ((g * jax.lax.logistic(g)) * (c3 + corr3)).astype(bf)
        hs = jnp.dot(u, w2_ref[...], preferred_element_type=jnp.float32)
        y2 = jnp.dot(hs.astype(bf), a2t_ref[...], preferred_element_type=jnp.float32)
        y2m = (y2 * m).astype(bf)
        hs = hs + jnp.dot(y2m, b2_ref[...], preferred_element_type=jnp.float32) * SCALE
        acc = acc + hs * wt_ref[...][:, :1]
    out_ref[...] = acc


def kernel(data, mask, rope_cos, rope_sin, attn_norm_w, ffn_norm_w, wq, wk, wv,
           wo, gate_w, w1, w2, w3, lora_a1, lora_b1, lora_a3, lora_b3,
           lora_a2, lora_b2):
    x = data[0]
    anw = attn_norm_w.reshape(1, D)
    fnw = ffn_norm_w.reshape(1, D)
    bf = jnp.bfloat16
    a1t = lora_a1.reshape(ER, FF).T.astype(bf)
    a3t = lora_a3.reshape(ER, FF).T.astype(bf)
    b1 = lora_b1.transpose(0, 2, 1).reshape(ER, FF).astype(bf)
    b3 = lora_b3.transpose(0, 2, 1).reshape(ER, FF).astype(bf)
    a2t = lora_a2.reshape(ER, D).T.astype(bf)
    b2 = lora_b2.transpose(0, 2, 1).reshape(ER, D).astype(bf)
    wqb = wq.astype(bf)
    wkb = wk.astype(bf)
    wvb = wv.astype(bf)
    wob = wo.astype(bf)
    w1b = w1.astype(bf)
    w2b = w2.astype(bf)
    w3b = w3.astype(bf)
    f32 = jnp.float32

    full = lambda shape: pl.BlockSpec(shape, lambda i: (0,) * len(shape))
    rows = lambda cols: pl.BlockSpec((SBLK, cols), lambda i: (i, 0))

    q3, k3, v3 = pl.pallas_call(
        _qkv_kernel,
        grid=(NBLK,),
        in_specs=[rows(D), rows(DH), rows(DH), full((1, D)),
                  full((D, H * DH)), full((D, KVH * DH)), full((D, KVH * DH))],
        out_specs=[pl.BlockSpec((H, SBLK, DH), lambda i: (0, i, 0)),
                   pl.BlockSpec((KVH, SBLK, DH), lambda i: (0, i, 0)),
                   pl.BlockSpec((KVH, SBLK, DH), lambda i: (0, i, 0))],
        out_shape=[jax.ShapeDtypeStruct((H, S, DH), bf),
                   jax.ShapeDtypeStruct((KVH, S, DH), bf),
                   jax.ShapeDtypeStruct((KVH, S, DH), bf)],
    )(x, rope_cos, rope_sin, anw, wqb, wkb, wvb)

    attn3 = pl.pallas_call(
        _attn_kernel,
        grid=(H, NABLK),
        in_specs=[pl.BlockSpec((1, ABLK, DH), lambda h, i: (h, i, 0)),
                  pl.BlockSpec((1, S, DH), lambda h, i: (h // NREP, 0, 0)),
                  pl.BlockSpec((1, S, DH), lambda h, i: (h // NREP, 0, 0)),
                  pl.BlockSpec((S, S), lambda h, i: (0, 0))],
        out_specs=pl.BlockSpec((1, ABLK, DH), lambda h, i: (h, i, 0)),
        out_shape=jax.ShapeDtypeStruct((H, S, DH), bf),
        compiler_params=pltpu.CompilerParams(vmem_limit_bytes=100 * 1024 * 1024),
    )(q3, k3, v3, mask * jnp.float32(LOG2E))

    d2, sn, lg = pl.pallas_call(
        _post_kernel,
        grid=(NBLK,),
        in_specs=[pl.BlockSpec((H, SBLK, DH), lambda i: (0, i, 0)),
                  rows(D), full((H * DH, D)), full((1, D)), full((D, E))],
        out_specs=[rows(D), rows(D), rows(E)],
        out_shape=[jax.ShapeDtypeStruct((S, D), f32),
                   jax.ShapeDtypeStruct((S, D), f32),
                   jax.ShapeDtypeStruct((S, E), f32)],
    )(attn3, x, wob, fnw, gate_w)

    oh0, oh1, wt0, wt1 = pl.pallas_call(
        _route_kernel,
        out_shape=[jax.ShapeDtypeStruct((S, E), f32)] * 4,
    )(lg)

    out = pl.pallas_call(
        _moe_kernel,
        grid=(NBLK,),
        in_specs=[rows(D), rows(D), rows(E), rows(E), rows(E), rows(E),
                  full((D, FF)), full((D, FF)), full((FF, D)),
                  full((FF, ER)), full((FF, ER)), full((ER, FF)),
                  full((ER, FF)), full((D, ER)), full((ER, D))],
        out_specs=rows(D),
        out_shape=jax.ShapeDtypeStruct((S, D), f32),
        compiler_params=pltpu.CompilerParams(vmem_limit_bytes=100 * 1024 * 1024),
    )(d2, sn, oh0, oh1, wt0, wt1, w1b, w3b, w2b, a1t, a3t, b1, b3, a2t, b2)

    return out[None]
